# baseline (speedup 1.0000x reference)
import jax
import jax.numpy as jnp
from jax.experimental import pallas as pl
from jax.experimental.pallas import tpu as pltpu

N_PERIOD = 4
N_STATIONS = 162
N_PLATFORMS = 460
N_CONT = 7
N_FEAT = 12
LANES = 128

ROW_PLAT = 0
ROW_STAT = 4
N_TAB_ROWS = 16


def _bl_kernel(x_ref, y_ref, tab_ref, tp_ref, wn_ref, wy_ref, by_ref,
               out_ref):
    bs = x_ref.shape[1]
    i32 = jnp.int32
    f32 = jnp.float32

    def gather_row(r, idx):
        row = jnp.broadcast_to(tab_ref[r:r + 1, :], (bs, LANES))
        return jnp.take_along_axis(row, idx, axis=1, mode="promise_in_bounds")

    idx_p = x_ref[0].astype(i32)
    acc = jnp.full((bs, LANES), tp_ref[0], f32)
    for r in range(1, N_PERIOD):
        acc = jnp.where(idx_p == r, tp_ref[r], acc)

    for s in range(3):
        idx = x_ref[1 + s].astype(i32)
        lo = jnp.bitwise_and(idx, LANES - 1)
        g0 = gather_row(ROW_STAT + 2 * s, lo)
        g1 = gather_row(ROW_STAT + 2 * s + 1, lo)
        acc = acc + jnp.where(idx < LANES, g0, g1)

    idx = x_ref[4].astype(i32)
    lo = jnp.bitwise_and(idx, LANES - 1)
    hi = jnp.right_shift(idx, 7)
    gp = gather_row(ROW_PLAT, lo)
    for c in range(1, 4):
        gp = jnp.where(hi == c, gather_row(ROW_PLAT + c, lo), gp)
    acc = acc + gp

    for k in range(N_CONT):
        acc = acc + wn_ref[k] * x_ref[5 + k]

    acc = acc + y_ref[...] * wy_ref[0] + by_ref[0]

    out_ref[...] = acc


def kernel(emb_period, emb_stations, emb_platforms, w_period, w_stations,
           w_platforms, w_fcn, w_fcy, b_fcy, x, y):
    if x.ndim == 1:
        x = x.reshape(1, -1)
    n = x.shape[0]
    bs = 512
    blk = bs * LANES
    n_pad = ((n + blk - 1) // blk) * blk
    rows = n_pad // LANES
    f32 = jnp.float32

    tp = (emb_period @ w_period.T).reshape(N_PERIOD)
    ts = jnp.stack(
        [(emb_stations @ w_stations[:, 3 * c:3 * c + 3].T)[:, 0]
         for c in range(3)], axis=0)
    tpl = (emb_platforms @ w_platforms.T).reshape(N_PLATFORMS)

    tab = jnp.zeros((N_TAB_ROWS, LANES), f32)
    tab = tab.at[ROW_PLAT:ROW_PLAT + 4, :].set(
        jnp.pad(tpl, (0, 4 * LANES - N_PLATFORMS)).reshape(4, LANES))
    tab = tab.at[ROW_STAT:ROW_STAT + 6, :].set(
        jnp.pad(ts, ((0, 0), (0, 2 * LANES - N_STATIONS))).reshape(6, LANES))

    wn = w_fcn.reshape(N_CONT).astype(f32)
    wy = w_fcy.reshape(1).astype(f32)
    by = b_fcy.reshape(1).astype(f32)

    smem = pl.BlockSpec(memory_space=pltpu.MemorySpace.SMEM)

    n_chunks = 4 if n_pad % (4 * blk) == 0 and n_pad >= 4 * blk else 1
    cn = n_pad // n_chunks
    crows = cn // LANES
    grid = (cn // blk,)

    xp = jnp.pad(x.astype(f32), ((0, n_pad - n), (0, 0)))
    yp = jnp.pad(y.reshape(n).astype(f32), (0, n_pad - n))

    outs = []
    for c in range(n_chunks):
        xt = xp[c * cn:(c + 1) * cn].T.reshape(N_FEAT, crows, LANES)
        yt = yp[c * cn:(c + 1) * cn].reshape(crows, LANES)
        outs.append(pl.pallas_call(
            _bl_kernel,
            out_shape=jax.ShapeDtypeStruct((crows, LANES), f32),
            grid=grid,
            in_specs=[
                pl.BlockSpec((N_FEAT, bs, LANES), lambda i: (0, i, 0)),
                pl.BlockSpec((bs, LANES), lambda i: (i, 0)),
                pl.BlockSpec((N_TAB_ROWS, LANES), lambda i: (0, 0)),
                smem,
                smem,
                smem,
                smem,
            ],
            out_specs=pl.BlockSpec((bs, LANES), lambda i: (i, 0)),
            compiler_params=pltpu.CompilerParams(
                dimension_semantics=("parallel",),
                vmem_limit_bytes=64 * 1024 * 1024),
        )(xt, yt, tab, tp.astype(f32), wn, wy, by))

    out = jnp.concatenate(outs, axis=0)
    return out.reshape(-1)[:n].reshape(n, 1)


# speedup vs baseline: 1.4578x; 1.4578x over previous
import jax
import jax.numpy as jnp
from jax.experimental import pallas as pl
from jax.experimental.pallas import tpu as pltpu

N_PERIOD = 4
N_STATIONS = 162
N_PLATFORMS = 460
N_CONT = 7
N_FEAT = 12
LANES = 128

ROW_PLAT = 0
ROW_STAT = 4
N_TAB_ROWS = 16


def _bl_kernel(x_ref, y_ref, tab_ref, tp_ref, wn_ref, wy_ref, by_ref,
               out_ref):
    bs = x_ref.shape[1]
    i32 = jnp.int32
    f32 = jnp.float32

    def gather_row(r, idx):
        row = jnp.broadcast_to(tab_ref[r:r + 1, :], (bs, LANES))
        return jnp.take_along_axis(row, idx, axis=1, mode="promise_in_bounds")

    idx_p = x_ref[0].astype(i32)
    acc = jnp.full((bs, LANES), tp_ref[0], f32)
    for r in range(1, N_PERIOD):
        acc = jnp.where(idx_p == r, tp_ref[r], acc)

    for s in range(3):
        idx = x_ref[1 + s].astype(i32)
        lo = jnp.bitwise_and(idx, LANES - 1)
        g0 = gather_row(ROW_STAT + 2 * s, lo)
        g1 = gather_row(ROW_STAT + 2 * s + 1, lo)
        acc = acc + jnp.where(idx < LANES, g0, g1)

    idx = x_ref[4].astype(i32)
    lo = jnp.bitwise_and(idx, LANES - 1)
    hi = jnp.right_shift(idx, 7)
    gp = gather_row(ROW_PLAT, lo)
    for c in range(1, 4):
        gp = jnp.where(hi == c, gather_row(ROW_PLAT + c, lo), gp)
    acc = acc + gp

    for k in range(N_CONT):
        acc = acc + wn_ref[k] * x_ref[5 + k]

    acc = acc + y_ref[...] * wy_ref[0] + by_ref[0]

    out_ref[...] = acc


def kernel(emb_period, emb_stations, emb_platforms, w_period, w_stations,
           w_platforms, w_fcn, w_fcy, b_fcy, x, y):
    if x.ndim == 1:
        x = x.reshape(1, -1)
    n = x.shape[0]
    bs = 512
    blk = bs * LANES
    n_pad = ((n + blk - 1) // blk) * blk
    rows = n_pad // LANES
    f32 = jnp.float32

    tp = (emb_period @ w_period.T).reshape(N_PERIOD)
    ts = jnp.stack(
        [(emb_stations @ w_stations[:, 3 * c:3 * c + 3].T)[:, 0]
         for c in range(3)], axis=0)
    tpl = (emb_platforms @ w_platforms.T).reshape(N_PLATFORMS)

    tab = jnp.zeros((N_TAB_ROWS, LANES), f32)
    tab = tab.at[ROW_PLAT:ROW_PLAT + 4, :].set(
        jnp.pad(tpl, (0, 4 * LANES - N_PLATFORMS)).reshape(4, LANES))
    tab = tab.at[ROW_STAT:ROW_STAT + 6, :].set(
        jnp.pad(ts, ((0, 0), (0, 2 * LANES - N_STATIONS))).reshape(6, LANES))

    wn = w_fcn.reshape(N_CONT).astype(f32)
    wy = w_fcy.reshape(1).astype(f32)
    by = b_fcy.reshape(1).astype(f32)

    smem = pl.BlockSpec(memory_space=pltpu.MemorySpace.SMEM)
    grid = (n_pad // blk,)

    xt = jnp.pad(x.astype(f32).T, ((0, 0), (0, n_pad - n)))
    xt = xt.reshape(N_FEAT, rows, LANES)
    yt = jnp.pad(y.reshape(n).astype(f32), (0, n_pad - n)).reshape(rows, LANES)

    out = pl.pallas_call(
        _bl_kernel,
        out_shape=jax.ShapeDtypeStruct((rows, LANES), f32),
        grid=grid,
        in_specs=[
            pl.BlockSpec((N_FEAT, bs, LANES), lambda i: (0, i, 0)),
            pl.BlockSpec((bs, LANES), lambda i: (i, 0)),
            pl.BlockSpec((N_TAB_ROWS, LANES), lambda i: (0, 0)),
            smem,
            smem,
            smem,
            smem,
        ],
        out_specs=pl.BlockSpec((bs, LANES), lambda i: (i, 0)),
        compiler_params=pltpu.CompilerParams(
            dimension_semantics=("parallel",),
            vmem_limit_bytes=64 * 1024 * 1024),
    )(xt, yt, tab, tp.astype(f32), wn, wy, by)

    return out.reshape(-1)[:n].reshape(n, 1)


# speedup vs baseline: 1.4738x; 1.0109x over previous
import jax
import jax.numpy as jnp
from jax.experimental import pallas as pl
from jax.experimental.pallas import tpu as pltpu

N_PERIOD = 4
N_STATIONS = 162
N_PLATFORMS = 460
N_CONT = 7
N_FEAT = 12
LANES = 128

ROW_PLAT = 0
ROW_STAT = 4
N_TAB_ROWS = 16


def _bl_kernel(x_ref, y_ref, tab_ref, tp_ref, wn_ref, wy_ref, by_ref,
               out_ref):
    bs = x_ref.shape[1]
    i32 = jnp.int32
    f32 = jnp.float32

    def gather_row(r, idx):
        row = jnp.broadcast_to(tab_ref[r:r + 1, :], (bs, LANES))
        return jnp.take_along_axis(row, idx, axis=1, mode="promise_in_bounds")

    idx_p = x_ref[0].astype(i32)
    acc = jnp.full((bs, LANES), tp_ref[0], f32)
    for r in range(1, N_PERIOD):
        acc = jnp.where(idx_p == r, tp_ref[r], acc)

    for s in range(3):
        idx = x_ref[1 + s].astype(i32)
        lo = jnp.bitwise_and(idx, LANES - 1)
        g0 = gather_row(ROW_STAT + 2 * s, lo)
        g1 = gather_row(ROW_STAT + 2 * s + 1, lo)
        acc = acc + jnp.where(idx < LANES, g0, g1)

    idx = x_ref[4].astype(i32)
    lo = jnp.bitwise_and(idx, LANES - 1)
    hi = jnp.right_shift(idx, 7)
    gp = gather_row(ROW_PLAT, lo)
    for c in range(1, 4):
        gp = jnp.where(hi == c, gather_row(ROW_PLAT + c, lo), gp)
    acc = acc + gp

    for k in range(N_CONT):
        acc = acc + wn_ref[k] * x_ref[5 + k]

    acc = acc + y_ref[...] * wy_ref[0] + by_ref[0]

    out_ref[...] = acc


def kernel(emb_period, emb_stations, emb_platforms, w_period, w_stations,
           w_platforms, w_fcn, w_fcy, b_fcy, x, y):
    if x.ndim == 1:
        x = x.reshape(1, -1)
    n = x.shape[0]
    bs = 1024
    blk = bs * LANES
    n_pad = ((n + blk - 1) // blk) * blk
    rows = n_pad // LANES
    f32 = jnp.float32

    tp = (emb_period @ w_period.T).reshape(N_PERIOD)
    ts = jnp.stack(
        [(emb_stations @ w_stations[:, 3 * c:3 * c + 3].T)[:, 0]
         for c in range(3)], axis=0)
    tpl = (emb_platforms @ w_platforms.T).reshape(N_PLATFORMS)

    tab = jnp.zeros((N_TAB_ROWS, LANES), f32)
    tab = tab.at[ROW_PLAT:ROW_PLAT + 4, :].set(
        jnp.pad(tpl, (0, 4 * LANES - N_PLATFORMS)).reshape(4, LANES))
    tab = tab.at[ROW_STAT:ROW_STAT + 6, :].set(
        jnp.pad(ts, ((0, 0), (0, 2 * LANES - N_STATIONS))).reshape(6, LANES))

    wn = w_fcn.reshape(N_CONT).astype(f32)
    wy = w_fcy.reshape(1).astype(f32)
    by = b_fcy.reshape(1).astype(f32)

    smem = pl.BlockSpec(memory_space=pltpu.MemorySpace.SMEM)
    grid = (n_pad // blk,)

    xt = jnp.pad(x.astype(f32).T, ((0, 0), (0, n_pad - n)))
    xt = xt.reshape(N_FEAT, rows, LANES)
    yt = jnp.pad(y.reshape(n).astype(f32), (0, n_pad - n)).reshape(rows, LANES)

    out = pl.pallas_call(
        _bl_kernel,
        out_shape=jax.ShapeDtypeStruct((rows, LANES), f32),
        grid=grid,
        in_specs=[
            pl.BlockSpec((N_FEAT, bs, LANES), lambda i: (0, i, 0)),
            pl.BlockSpec((bs, LANES), lambda i: (i, 0)),
            pl.BlockSpec((N_TAB_ROWS, LANES), lambda i: (0, 0)),
            smem,
            smem,
            smem,
            smem,
        ],
        out_specs=pl.BlockSpec((bs, LANES), lambda i: (i, 0)),
        compiler_params=pltpu.CompilerParams(
            dimension_semantics=("parallel",),
            vmem_limit_bytes=64 * 1024 * 1024),
    )(xt, yt, tab, tp.astype(f32), wn, wy, by)

    return out.reshape(-1)[:n].reshape(n, 1)
